# baseline (speedup 1.0000x reference)
import jax
import jax.numpy as jnp
from jax import lax
from jax.experimental import pallas as pl
from jax.experimental.pallas import tpu as pltpu

ROWS = 128
COLS = 32768
K = 64
CHUNK = 2048
NCH = COLS // CHUNK
NBUF = 8


def _body(out_hbm, y_hbm, res_ref, ob, yb, osem, ysem):
    def start(c):
        b = c % NBUF
        col = c * CHUNK
        pltpu.make_async_copy(out_hbm.at[:, pl.ds(col, CHUNK)], ob.at[b],
                              osem.at[b]).start()
        pltpu.make_async_copy(y_hbm.at[:, pl.ds(col, CHUNK)], yb.at[b],
                              ysem.at[b]).start()

    def wait(c):
        b = c % NBUF
        col = c * CHUNK
        pltpu.make_async_copy(out_hbm.at[:, pl.ds(col, CHUNK)], ob.at[b],
                              osem.at[b]).wait()
        pltpu.make_async_copy(y_hbm.at[:, pl.ds(col, CHUNK)], yb.at[b],
                              ysem.at[b]).wait()

    for c in range(NBUF):
        start(c)
    acc = jnp.zeros((ROWS, 1), jnp.float32)
    for c in range(NCH):
        wait(c)
        b = c % NBUF
        d = ob[b] - yb[b]
        acc = acc + jnp.sum(d * d, axis=1, keepdims=True)
        if c + NBUF < NCH:
            start(c + NBUF)

    v = acc * (1.0 / COLS)
    eye = (lax.broadcasted_iota(jnp.int32, (ROWS, ROWS), 0) ==
           lax.broadcasted_iota(jnp.int32, (ROWS, ROWS), 1)).astype(jnp.float32)
    vrow = lax.dot_general(v, eye, (((0,), (0,)), ((), ())),
                           preferred_element_type=jnp.float32)
    gt = (vrow > v).astype(jnp.float32)
    ones = jnp.ones((ROWS, 1), jnp.float32)
    rank = lax.dot_general(gt, ones, (((1,), (0,)), ((), ())),
                           preferred_element_type=jnp.float32)
    cand = rank < K
    t = jnp.min(jnp.where(cand, v, jnp.inf))
    above = v > t
    n_above = jnp.sum(above.astype(jnp.float32))
    s_above = jnp.sum(jnp.where(above, v, 0.0))
    total = s_above + t * (K - n_above)
    res_ref[...] = total.reshape(1, 1)


def kernel(out, y):
    res = pl.pallas_call(
        _body,
        in_specs=[
            pl.BlockSpec(memory_space=pltpu.MemorySpace.HBM),
            pl.BlockSpec(memory_space=pltpu.MemorySpace.HBM),
        ],
        out_shape=jax.ShapeDtypeStruct((1, 1), jnp.float32),
        scratch_shapes=[
            pltpu.VMEM((NBUF, ROWS, CHUNK), jnp.float32),
            pltpu.VMEM((NBUF, ROWS, CHUNK), jnp.float32),
            pltpu.SemaphoreType.DMA((NBUF,)),
            pltpu.SemaphoreType.DMA((NBUF,)),
        ],
    )(out, y)
    return res[0, 0]


# speedup vs baseline: 1.0224x; 1.0224x over previous
import jax
import jax.numpy as jnp
from jax import lax
from jax.experimental import pallas as pl
from jax.experimental.pallas import tpu as pltpu

ROWS = 128
COLS = 32768
K = 64
CHUNK = 4096
NCH = COLS // CHUNK
NBUF = 4


def _body(out_hbm, y_hbm, res_ref, ob, yb, osem, ysem):
    def start(c):
        b = c % NBUF
        col = c * CHUNK
        pltpu.make_async_copy(out_hbm.at[:, pl.ds(col, CHUNK)], ob.at[b],
                              osem.at[b]).start()
        pltpu.make_async_copy(y_hbm.at[:, pl.ds(col, CHUNK)], yb.at[b],
                              ysem.at[b]).start()

    def wait(c):
        b = c % NBUF
        col = c * CHUNK
        pltpu.make_async_copy(out_hbm.at[:, pl.ds(col, CHUNK)], ob.at[b],
                              osem.at[b]).wait()
        pltpu.make_async_copy(y_hbm.at[:, pl.ds(col, CHUNK)], yb.at[b],
                              ysem.at[b]).wait()

    for c in range(NBUF):
        start(c)
    acc = jnp.zeros((ROWS, 1), jnp.float32)
    for c in range(NCH):
        wait(c)
        b = c % NBUF
        d = ob[b] - yb[b]
        acc = acc + jnp.sum(d * d, axis=1, keepdims=True)
        if c + NBUF < NCH:
            start(c + NBUF)

    v = acc * (1.0 / COLS)
    eye = (lax.broadcasted_iota(jnp.int32, (ROWS, ROWS), 0) ==
           lax.broadcasted_iota(jnp.int32, (ROWS, ROWS), 1)).astype(jnp.float32)
    vrow = lax.dot_general(v, eye, (((0,), (0,)), ((), ())),
                           preferred_element_type=jnp.float32)
    gt = (vrow > v).astype(jnp.float32)
    ones = jnp.ones((ROWS, 1), jnp.float32)
    rank = lax.dot_general(gt, ones, (((1,), (0,)), ((), ())),
                           preferred_element_type=jnp.float32)
    cand = rank < K
    t = jnp.min(jnp.where(cand, v, jnp.inf))
    above = v > t
    n_above = jnp.sum(above.astype(jnp.float32))
    s_above = jnp.sum(jnp.where(above, v, 0.0))
    total = s_above + t * (K - n_above)
    res_ref[...] = total.reshape(1, 1)


def kernel(out, y):
    res = pl.pallas_call(
        _body,
        in_specs=[
            pl.BlockSpec(memory_space=pltpu.MemorySpace.HBM),
            pl.BlockSpec(memory_space=pltpu.MemorySpace.HBM),
        ],
        out_shape=jax.ShapeDtypeStruct((1, 1), jnp.float32),
        scratch_shapes=[
            pltpu.VMEM((NBUF, ROWS, CHUNK), jnp.float32),
            pltpu.VMEM((NBUF, ROWS, CHUNK), jnp.float32),
            pltpu.SemaphoreType.DMA((NBUF,)),
            pltpu.SemaphoreType.DMA((NBUF,)),
        ],
    )(out, y)
    return res[0, 0]
